# baseline (speedup 1.0000x reference)
import functools

import jax
import jax.numpy as jnp
from jax import lax
from jax.experimental import pallas as pl
from jax.experimental.pallas import tpu as pltpu
from jax.experimental.pallas import tpu_sc as plsc

VOCAB = 28996
DIM = 768
SEQ = 64
BATCH = 4096

NUM_CORES = 2
NUM_SUBCORES = 16
NUM_WORKERS = NUM_CORES * NUM_SUBCORES

B_TOTAL = BATCH * SEQ
B_PER_W = B_TOTAL // NUM_WORKERS
CHUNK = SEQ
N_CHUNKS = B_PER_W // CHUNK
LANES = 16
COLS = DIM // LANES


def _build_kernel():
    mesh = plsc.VectorSubcoreMesh(core_axis_name="c", subcore_axis_name="s")

    @functools.partial(
        pl.kernel,
        mesh=mesh,
        out_type=jax.ShapeDtypeStruct((B_TOTAL, DIM), jnp.float32),
        scratch_types=[
            pltpu.VMEM((N_CHUNKS, CHUNK), jnp.int32),
            pltpu.VMEM((SEQ, DIM), jnp.float32),
            pltpu.VMEM((CHUNK, DIM), jnp.float32),
            pltpu.SemaphoreType.DMA,
        ],
    )
    def emb_kernel(x_hbm, table_hbm, pos_hbm, out_hbm, idx_v, pos_v, rows_v,
                   gsem):
        wid = lax.axis_index("s") * NUM_CORES + lax.axis_index("c")
        base = wid * B_PER_W
        pltpu.sync_copy(x_hbm.at[wid], idx_v)
        pltpu.sync_copy(pos_hbm, pos_v)

        def chunk_body(j, carry):
            pltpu.async_copy(table_hbm.at[idx_v.at[j]], rows_v, gsem).wait()

            def row_body(r, c2):
                for c in range(COLS):
                    sl = pl.ds(c * LANES, LANES)
                    rows_v[r, sl] = rows_v[r, sl] + pos_v[r, sl]
                return c2

            lax.fori_loop(0, CHUNK, row_body, 0, unroll=False)
            pltpu.sync_copy(rows_v, out_hbm.at[pl.ds(base + j * CHUNK, CHUNK)])
            return carry

        lax.fori_loop(0, N_CHUNKS, chunk_body, 0, unroll=False)

    return emb_kernel


_EMB_KERNEL = None


def kernel(x, wte, pos_emb):
    global _EMB_KERNEL
    if _EMB_KERNEL is None:
        _EMB_KERNEL = _build_kernel()
    seq_len = x.shape[1]
    x3 = x.astype(jnp.int32).reshape(NUM_WORKERS, N_CHUNKS, CHUNK)
    pos = pos_emb[:seq_len, :].astype(jnp.float32)
    out = _EMB_KERNEL(x3, wte, pos)
    return out.reshape(BATCH, SEQ, DIM)
